# baseline (speedup 1.0000x reference)
import functools

import jax
import jax.numpy as jnp
import numpy as np
from jax import lax
from jax.experimental import pallas as pl
from jax.experimental.pallas import tpu as pltpu
from jax.experimental.pallas import tpu_sc as plsc

NUM_BUCKETS = 1000000
EMB_DIM = 32

_B = 16384
_C = 200
_SLAB = 512
_NSTREAM = _SLAB // 128

_info = plsc.get_sparse_core_info()
_NC, _NS = _info.num_cores, _info.num_subcores
_NW = _NC * _NS
assert _B // _NW == _SLAB

_D = NUM_BUCKETS - 1
_RECIP = np.float32(1.0 / _D)


def _hash16(v):
    q = (v.astype(jnp.float32) * _RECIP).astype(jnp.int32)
    r = v - q * _D
    r = jnp.where(r < 0, r + _D, r)
    r = jnp.where(r >= _D, r - _D, r)
    return jnp.where(v == 0, 0, r + 1)


def _sc_body(xt_hbm, table_hbm, out_hbm,
             idx0, idx1, rows0, rows1, t0, t1, sem0, sem1):
    idx = (idx0, idx1)
    rows = (rows0, rows1)
    tb = (t0, t1)
    sem = (sem0, sem1)
    wid = lax.axis_index("s") * _NC + lax.axis_index("c")
    b0 = wid * _SLAB
    iota16 = lax.iota(jnp.int32, 16)

    def prep(c, b):
        pltpu.sync_copy(xt_hbm.at[c, pl.ds(b0, _SLAB)], idx[b])

        def hash_body(i, carry):
            off = i * 64
            for j in range(4):
                s = pl.ds(off + j * 16, 16)
                idx[b][s] = _hash16(idx[b][s])
            return carry

        lax.fori_loop(0, _SLAB // 64, hash_body, 0)
        for j in range(_NSTREAM):
            pltpu.async_copy(
                table_hbm.at[idx[b].at[pl.ds(128 * j, 128)]],
                rows[b].at[pl.ds(128 * j, 128)], sem[b])

    def finish(c, b):
        for j in range(_NSTREAM):
            pltpu.make_async_copy(
                table_hbm.at[idx[b].at[pl.ds(128 * j, 128)]],
                rows[b].at[pl.ds(128 * j, 128)], sem[b]).wait()

        for dblk in range(4):
            def trans_body(ds, carry):
                d = dblk * 8 + ds
                dvec = jnp.full((16,), 1, jnp.int32) * d
                base = ds * 128
                for bb in range(4):
                    for g in range(8):
                        row_idx = iota16 + (bb * 128 + g * 16)
                        v = plsc.load_gather(rows[b], [row_idx, dvec])
                        tb[b][dblk, pl.ds(base + bb * 1024 + g * 16, 16)] = v
                return carry

            lax.fori_loop(0, 8, trans_body, 0)

        pltpu.sync_copy(
            tb[b], out_hbm.at[pl.ds(c * 4, 4), pl.ds(4096 * wid, 4096)])

    prep(0, 0)

    def loop(p, carry):
        c0 = 2 * p

        @pl.when(c0 + 1 < _C)
        def _():
            prep(c0 + 1, 1)

        finish(c0, 0)

        @pl.when(c0 + 2 < _C)
        def _():
            prep(c0 + 2, 0)

        finish(c0 + 1, 1)
        return carry

    lax.fori_loop(0, _C // 2, loop, 0)


@jax.jit
def kernel(x, table):
    xt = x.T
    run = functools.partial(
        pl.kernel,
        mesh=plsc.VectorSubcoreMesh(core_axis_name="c", subcore_axis_name="s"),
        out_type=jax.ShapeDtypeStruct((_C * 4, _B * 8), jnp.float32),
        scratch_types=[
            pltpu.VMEM((_SLAB,), jnp.int32),
            pltpu.VMEM((_SLAB,), jnp.int32),
            pltpu.VMEM((_SLAB, EMB_DIM), jnp.float32),
            pltpu.VMEM((_SLAB, EMB_DIM), jnp.float32),
            pltpu.VMEM((4, _SLAB * 8), jnp.float32),
            pltpu.VMEM((4, _SLAB * 8), jnp.float32),
            pltpu.SemaphoreType.DMA,
            pltpu.SemaphoreType.DMA,
        ],
        compiler_params=pltpu.CompilerParams(
            use_tc_tiling_on_sc=False, needs_layout_passes=False),
    )(_sc_body)
    out = run(xt, table)
    return (out.reshape(_C, 4, 128, 8, 128)
            .transpose(2, 4, 0, 1, 3)
            .reshape(_B, _C, EMB_DIM))


# speedup vs baseline: 1.6788x; 1.6788x over previous
import functools

import jax
import jax.numpy as jnp
import numpy as np
from jax import lax
from jax.experimental import pallas as pl
from jax.experimental.pallas import tpu as pltpu
from jax.experimental.pallas import tpu_sc as plsc

NUM_BUCKETS = 1000000
EMB_DIM = 32

_B = 16384
_C = 200
_SLAB = 512
_NSTREAM = _SLAB // 128
_ROWLEN = _B * 8
_SEG = _SLAB * 8

_info = plsc.get_sparse_core_info()
_NC, _NS = _info.num_cores, _info.num_subcores
_NW = _NC * _NS
assert _B // _NW == _SLAB

_D = NUM_BUCKETS - 1
_RECIP = np.float32(1.0 / _D)


def _hash16(v):
    q = (v.astype(jnp.float32) * _RECIP).astype(jnp.int32)
    r = v - q * _D
    r = jnp.where(r < 0, r + _D, r)
    r = jnp.where(r >= _D, r - _D, r)
    return jnp.where(v == 0, 0, r + 1)


def _sc_body(xt_hbm, table_hbm, out_hbm,
             idx0, idx1, rows0, rows1, t0, t1,
             isem0, isem1, gsem0, gsem1, wsem0, wsem1):
    idx = (idx0, idx1)
    rows = (rows0, rows1)
    tb = (t0, t1)
    isem = (isem0, isem1)
    gsem = (gsem0, gsem1)
    wsem = (wsem0, wsem1)
    wid = lax.axis_index("s") * _NC + lax.axis_index("c")
    b0 = wid * _SLAB

    iota16 = lax.iota(jnp.int32, 16)
    dconst0 = (iota16 >> 3) * 4096 + (iota16 & 7) * 128
    dconst = (dconst0, dconst0 + 8192)

    def prep(c, b):
        pltpu.async_copy(xt_hbm.at[c, pl.ds(b0, _SLAB)], idx[b], isem[b])

    def work(c, b):
        pltpu.make_async_copy(
            xt_hbm.at[c, pl.ds(b0, _SLAB)], idx[b], isem[b]).wait()

        def hash_body(i, carry):
            off = i * 64
            for j in range(4):
                s = pl.ds(off + j * 16, 16)
                idx[b][s] = _hash16(idx[b][s])
            return carry

        lax.fori_loop(0, _SLAB // 64, hash_body, 0)
        for j in range(_NSTREAM):
            pltpu.async_copy(
                table_hbm.at[idx[b].at[pl.ds(128 * j, 128)]],
                rows[b].at[pl.ds(128 * j, 128)], gsem[b])

    def finish(c, b):
        for j in range(_NSTREAM):
            pltpu.make_async_copy(
                table_hbm.at[idx[b].at[pl.ds(128 * j, 128)]],
                rows[b].at[pl.ds(128 * j, 128)], gsem[b]).wait()

        @pl.when(c + 2 < _C)
        def _():
            prep(c + 2, b)

        @pl.when(c >= 2)
        def _():
            _drain_writes(c - 2, b)

        @plsc.parallel_loop(0, _SLAB // 4, unroll=2)
        def _(t):
            r0 = t * 4
            pos0 = (r0 >> 7) * 1024 + (r0 & 127)
            for k in range(4):
                for h in range(2):
                    v = rows[b][r0 + k, pl.ds(16 * h, 16)]
                    plsc.store_scatter(tb[b], [dconst[h] + (pos0 + k)], v)

        for dblk in range(4):
            pltpu.async_copy(
                tb[b].at[pl.ds(dblk * 4096, 4096)],
                out_hbm.at[pl.ds((c * 4 + dblk) * _ROWLEN + _SEG * wid, 4096)],
                wsem[b])

    def _drain_writes(c, b):
        for dblk in range(4):
            pltpu.make_async_copy(
                tb[b].at[pl.ds(dblk * 4096, 4096)],
                out_hbm.at[pl.ds((c * 4 + dblk) * _ROWLEN + _SEG * wid, 4096)],
                wsem[b]).wait()

    prep(0, 0)
    prep(1, 1)
    work(0, 0)

    def loop(p, carry):
        c0 = 2 * p
        work(c0 + 1, 1)
        finish(c0, 0)

        @pl.when(c0 + 2 < _C)
        def _():
            work(c0 + 2, 0)

        finish(c0 + 1, 1)
        return carry

    lax.fori_loop(0, _C // 2, loop, 0)
    _drain_writes(_C - 2, 0)
    _drain_writes(_C - 1, 1)


@jax.jit
def kernel(x, table):
    xt = x.T
    run = functools.partial(
        pl.kernel,
        mesh=plsc.VectorSubcoreMesh(core_axis_name="c", subcore_axis_name="s"),
        out_type=jax.ShapeDtypeStruct((_C * 4 * _ROWLEN,), jnp.float32),
        scratch_types=[
            pltpu.VMEM((_SLAB,), jnp.int32),
            pltpu.VMEM((_SLAB,), jnp.int32),
            pltpu.VMEM((_SLAB, EMB_DIM), jnp.float32),
            pltpu.VMEM((_SLAB, EMB_DIM), jnp.float32),
            pltpu.VMEM((4 * 4096,), jnp.float32),
            pltpu.VMEM((4 * 4096,), jnp.float32),
            pltpu.SemaphoreType.DMA,
            pltpu.SemaphoreType.DMA,
            pltpu.SemaphoreType.DMA,
            pltpu.SemaphoreType.DMA,
            pltpu.SemaphoreType.DMA,
            pltpu.SemaphoreType.DMA,
        ],
        compiler_params=pltpu.CompilerParams(
            use_tc_tiling_on_sc=False, needs_layout_passes=False),
    )(_sc_body)
    out = run(xt, table)
    return (out.reshape(_C, 4, 128, 8, 128)
            .transpose(2, 4, 0, 1, 3)
            .reshape(_B, _C, EMB_DIM))


# speedup vs baseline: 3.8909x; 2.3176x over previous
import functools

import jax
import jax.numpy as jnp
import numpy as np
from jax import lax
from jax.experimental import pallas as pl
from jax.experimental.pallas import tpu as pltpu
from jax.experimental.pallas import tpu_sc as plsc

NUM_BUCKETS = 1000000
EMB_DIM = 32

_B = 16384
_C = 200
_SLAB = 512
_NSTREAM = _SLAB // 128
_ROWLEN = _B * 8
_SEG = _SLAB * 8

_info = plsc.get_sparse_core_info()
_NC, _NS = _info.num_cores, _info.num_subcores
_NW = _NC * _NS
assert _B // _NW == _SLAB

_D = NUM_BUCKETS - 1
_RECIP = np.float32(1.0 / _D)


def _hash16(v):
    q = (v.astype(jnp.float32) * _RECIP).astype(jnp.int32)
    r = v - q * _D
    r = jnp.where(r < 0, r + _D, r)
    r = jnp.where(r >= _D, r - _D, r)
    return jnp.where(v == 0, 0, r + 1)


def _sc_body(xt_hbm, table_hbm, out_hbm,
             idx0, idx1, rows0, rows1, t0, t1,
             isem0, isem1, gsem0, gsem1, wsem0, wsem1):
    idx = (idx0, idx1)
    rows = (rows0, rows1)
    tb = (t0, t1)
    isem = (isem0, isem1)
    gsem = (gsem0, gsem1)
    wsem = (wsem0, wsem1)
    wid = lax.axis_index("s") * _NC + lax.axis_index("c")
    b0 = wid * _SLAB

    iota16 = lax.iota(jnp.int32, 16)

    def prep(c, b):
        pltpu.async_copy(xt_hbm.at[c, pl.ds(b0, _SLAB)], idx[b], isem[b])

    def work(c, b):
        pltpu.make_async_copy(
            xt_hbm.at[c, pl.ds(b0, _SLAB)], idx[b], isem[b]).wait()

        def hash_body(i, carry):
            off = i * 64
            for j in range(4):
                s = pl.ds(off + j * 16, 16)
                idx[b][s] = _hash16(idx[b][s])
            return carry

        lax.fori_loop(0, _SLAB // 64, hash_body, 0)
        for j in range(_NSTREAM):
            pltpu.async_copy(
                table_hbm.at[idx[b].at[pl.ds(128 * j, 128)]],
                rows[b].at[pl.ds(128 * j, 128)], gsem[b])

    def finish(c, b):
        for j in range(_NSTREAM):
            pltpu.make_async_copy(
                table_hbm.at[idx[b].at[pl.ds(128 * j, 128)]],
                rows[b].at[pl.ds(128 * j, 128)], gsem[b]).wait()

        @pl.when(c + 2 < _C)
        def _():
            prep(c + 2, b)

        @pl.when(c >= 2)
        def _():
            _drain_writes(c - 2, b)

        def skew_body(k, carry):
            dvec = (iota16 + k) & 15
            dpos = (dvec >> 3) * 4096 + (dvec & 7) * 128
            for h in range(2):
                cvec = dvec + 16 * h
                spv = dpos + 8192 * h + iota16
                @plsc.parallel_loop(0, _SLAB // 16, unroll=4)
                def _(rb):
                    r0 = rb * 16
                    row_idx = iota16 + r0
                    v = plsc.load_gather(rows[b], [row_idx, cvec])
                    pos0 = (r0 >> 7) * 1024 + (r0 & 127)
                    plsc.store_scatter(tb[b], [spv + pos0], v)
            return carry

        lax.fori_loop(0, 16, skew_body, 0)

        for dblk in range(4):
            pltpu.async_copy(
                tb[b].at[pl.ds(dblk * 4096, 4096)],
                out_hbm.at[pl.ds((c * 4 + dblk) * _ROWLEN + _SEG * wid, 4096)],
                wsem[b])

    def _drain_writes(c, b):
        for dblk in range(4):
            pltpu.make_async_copy(
                tb[b].at[pl.ds(dblk * 4096, 4096)],
                out_hbm.at[pl.ds((c * 4 + dblk) * _ROWLEN + _SEG * wid, 4096)],
                wsem[b]).wait()

    prep(0, 0)
    prep(1, 1)
    work(0, 0)

    def loop(p, carry):
        c0 = 2 * p
        work(c0 + 1, 1)
        finish(c0, 0)

        @pl.when(c0 + 2 < _C)
        def _():
            work(c0 + 2, 0)

        finish(c0 + 1, 1)
        return carry

    lax.fori_loop(0, _C // 2, loop, 0)
    _drain_writes(_C - 2, 0)
    _drain_writes(_C - 1, 1)


@jax.jit
def kernel(x, table):
    xt = x.T
    run = functools.partial(
        pl.kernel,
        mesh=plsc.VectorSubcoreMesh(core_axis_name="c", subcore_axis_name="s"),
        out_type=jax.ShapeDtypeStruct((_C * 4 * _ROWLEN,), jnp.float32),
        scratch_types=[
            pltpu.VMEM((_SLAB,), jnp.int32),
            pltpu.VMEM((_SLAB,), jnp.int32),
            pltpu.VMEM((_SLAB, EMB_DIM), jnp.float32),
            pltpu.VMEM((_SLAB, EMB_DIM), jnp.float32),
            pltpu.VMEM((4 * 4096,), jnp.float32),
            pltpu.VMEM((4 * 4096,), jnp.float32),
            pltpu.SemaphoreType.DMA,
            pltpu.SemaphoreType.DMA,
            pltpu.SemaphoreType.DMA,
            pltpu.SemaphoreType.DMA,
            pltpu.SemaphoreType.DMA,
            pltpu.SemaphoreType.DMA,
        ],
        compiler_params=pltpu.CompilerParams(
            use_tc_tiling_on_sc=False, needs_layout_passes=False),
    )(_sc_body)
    out = run(xt, table)
    return (out.reshape(_C, 4, 128, 8, 128)
            .transpose(2, 4, 0, 1, 3)
            .reshape(_B, _C, EMB_DIM))


# speedup vs baseline: 3.9255x; 1.0089x over previous
import functools

import jax
import jax.numpy as jnp
import numpy as np
from jax import lax
from jax.experimental import pallas as pl
from jax.experimental.pallas import tpu as pltpu
from jax.experimental.pallas import tpu_sc as plsc

NUM_BUCKETS = 1000000
EMB_DIM = 32

_B = 16384
_C = 200
_SLAB = 512
_NSTREAM = _SLAB // 128
_ROWLEN = _B * 8
_SEG = _SLAB * 8

_info = plsc.get_sparse_core_info()
_NC, _NS = _info.num_cores, _info.num_subcores
_NW = _NC * _NS
assert _B // _NW == _SLAB

_D = NUM_BUCKETS - 1
_RECIP = np.float32(1.0 / _D)


def _hash16(v):
    q = (v.astype(jnp.float32) * _RECIP).astype(jnp.int32)
    r = v - q * _D
    r = jnp.where(r < 0, r + _D, r)
    r = jnp.where(r >= _D, r - _D, r)
    return jnp.where(v == 0, 0, r + 1)


def _sc_body(xt_hbm, table_hbm, out_hbm,
             idx0, idx1, rows0, rows1, t0, t1,
             isem0, isem1, gsem0, gsem1, wsem0, wsem1):
    idx = (idx0, idx1)
    rows = (rows0, rows1)
    tb = (t0, t1)
    isem = (isem0, isem1)
    gsem = (gsem0, gsem1)
    wsem = (wsem0, wsem1)
    wid = lax.axis_index("s") * _NC + lax.axis_index("c")
    b0 = wid * _SLAB

    iota16 = lax.iota(jnp.int32, 16)

    def prep(c, b):
        pltpu.async_copy(xt_hbm.at[c, pl.ds(b0, _SLAB)], idx[b], isem[b])

    def work(c, b):
        pltpu.make_async_copy(
            xt_hbm.at[c, pl.ds(b0, _SLAB)], idx[b], isem[b]).wait()

        def hash_body(i, carry):
            off = i * 64
            for j in range(4):
                s = pl.ds(off + j * 16, 16)
                idx[b][s] = _hash16(idx[b][s])
            return carry

        lax.fori_loop(0, _SLAB // 64, hash_body, 0, unroll=2)
        for j in range(_NSTREAM):
            pltpu.async_copy(
                table_hbm.at[idx[b].at[pl.ds(128 * j, 128)]],
                rows[b].at[pl.ds(128 * j, 128)], gsem[b])

    def finish(c, b):
        for j in range(_NSTREAM):
            pltpu.make_async_copy(
                table_hbm.at[idx[b].at[pl.ds(128 * j, 128)]],
                rows[b].at[pl.ds(128 * j, 128)], gsem[b]).wait()

        @pl.when(c + 2 < _C)
        def _():
            prep(c + 2, b)

        @pl.when(c >= 2)
        def _():
            _drain_writes(c - 2, b)

        def skew_body(k, carry):
            dvec = (iota16 + k) & 15
            dpos = (dvec >> 3) * 4096 + (dvec & 7) * 128
            for h in range(2):
                cvec = dvec + 16 * h
                spv = dpos + 8192 * h + iota16
                @plsc.parallel_loop(0, _SLAB // 16, unroll=8)
                def _(rb):
                    r0 = rb * 16
                    row_idx = iota16 + r0
                    v = plsc.load_gather(rows[b], [row_idx, cvec])
                    pos0 = (r0 >> 7) * 1024 + (r0 & 127)
                    plsc.store_scatter(tb[b], [spv + pos0], v)
            return carry

        lax.fori_loop(0, 16, skew_body, 0)

        for dblk in range(4):
            pltpu.async_copy(
                tb[b].at[pl.ds(dblk * 4096, 4096)],
                out_hbm.at[pl.ds((c * 4 + dblk) * _ROWLEN + _SEG * wid, 4096)],
                wsem[b])

    def _drain_writes(c, b):
        for dblk in range(4):
            pltpu.make_async_copy(
                tb[b].at[pl.ds(dblk * 4096, 4096)],
                out_hbm.at[pl.ds((c * 4 + dblk) * _ROWLEN + _SEG * wid, 4096)],
                wsem[b]).wait()

    prep(0, 0)
    prep(1, 1)
    work(0, 0)

    def loop(p, carry):
        c0 = 2 * p
        work(c0 + 1, 1)
        finish(c0, 0)

        @pl.when(c0 + 2 < _C)
        def _():
            work(c0 + 2, 0)

        finish(c0 + 1, 1)
        return carry

    lax.fori_loop(0, _C // 2, loop, 0)
    _drain_writes(_C - 2, 0)
    _drain_writes(_C - 1, 1)


@jax.jit
def kernel(x, table):
    xt = x.T
    run = functools.partial(
        pl.kernel,
        mesh=plsc.VectorSubcoreMesh(core_axis_name="c", subcore_axis_name="s"),
        out_type=jax.ShapeDtypeStruct((_C * 4 * _ROWLEN,), jnp.float32),
        scratch_types=[
            pltpu.VMEM((_SLAB,), jnp.int32),
            pltpu.VMEM((_SLAB,), jnp.int32),
            pltpu.VMEM((_SLAB, EMB_DIM), jnp.float32),
            pltpu.VMEM((_SLAB, EMB_DIM), jnp.float32),
            pltpu.VMEM((4 * 4096,), jnp.float32),
            pltpu.VMEM((4 * 4096,), jnp.float32),
            pltpu.SemaphoreType.DMA,
            pltpu.SemaphoreType.DMA,
            pltpu.SemaphoreType.DMA,
            pltpu.SemaphoreType.DMA,
            pltpu.SemaphoreType.DMA,
            pltpu.SemaphoreType.DMA,
        ],
        compiler_params=pltpu.CompilerParams(
            use_tc_tiling_on_sc=False, needs_layout_passes=False),
    )(_sc_body)
    out = run(xt, table)
    return (out.reshape(_C, 4, 128, 8, 128)
            .transpose(2, 4, 0, 1, 3)
            .reshape(_B, _C, EMB_DIM))
